# Initial kernel scaffold; baseline (speedup 1.0000x reference)
import jax
import jax.numpy as jnp
from jax.experimental import pallas as pl


def kernel(x, W, b):
    raise NotImplementedError("write your pallas kernel here")


# speedup vs baseline: 1.6949x; 1.6949x over previous
import jax
import jax.numpy as jnp
from jax.experimental import pallas as pl

B = 128
IN = 64
N = 32768
K = 8
BLK1 = 2048
NB1 = N // BLK1
BLK2 = 4096
NB2 = N // BLK2
NEG = -1.0
IMAX = 2**31 - 1


def _p1(x_ref, w_ref, b_ref, y_ref, cv_ref, ci_ref):
    j = pl.program_id(0)
    xb = x_ref[...]
    wb = w_ref[...]
    yb = jax.lax.dot_general(xb, wb, (((1,), (1,)), ((), ())),
                             preferred_element_type=jnp.float32)
    y_ref[...] = yb + b_ref[...]
    x2 = jnp.sum(xb * xb, axis=1, keepdims=True)
    w2 = jnp.sum(wb * wb, axis=1)[None, :]
    s = jnp.sqrt(jnp.maximum(x2 + w2 - 2.0 * yb, 0.0))
    col = jax.lax.broadcasted_iota(jnp.int32, (B, BLK1), 1) + j * BLK1
    vals, idxs = [], []
    for _ in range(K):
        m = jnp.max(s, axis=1, keepdims=True)
        idx = jnp.min(jnp.where(s == m, col, IMAX), axis=1, keepdims=True)
        vals.append(m)
        idxs.append(idx)
        s = jnp.where(col == idx, NEG, s)
    cv_ref[0] = jnp.concatenate(vals, axis=1)
    ci_ref[0] = jnp.concatenate(idxs, axis=1)


def _p2(cv_ref, ci_ref, wta_ref):
    j = pl.program_id(0)
    v = cv_ref[...]
    i = ci_ref[...]
    tops = []
    for _ in range(K):
        m = jnp.max(v, axis=1, keepdims=True)
        idx = jnp.min(jnp.where(v == m, i, IMAX), axis=1, keepdims=True)
        tops.append(idx)
        v = jnp.where(i == idx, NEG, v)
    col = jax.lax.broadcasted_iota(jnp.int32, (B, BLK2), 1) + j * BLK2
    acc = col == tops[0]
    for t in tops[1:]:
        acc = acc | (col == t)
    wta_ref[...] = acc.astype(jnp.float32)


def kernel(x, W, b):
    b2 = b.reshape(1, N)
    y, cv, ci = pl.pallas_call(
        _p1,
        grid=(NB1,),
        in_specs=[
            pl.BlockSpec((B, IN), lambda j: (0, 0)),
            pl.BlockSpec((BLK1, IN), lambda j: (j, 0)),
            pl.BlockSpec((1, BLK1), lambda j: (0, j)),
        ],
        out_specs=[
            pl.BlockSpec((B, BLK1), lambda j: (0, j)),
            pl.BlockSpec((1, B, K), lambda j: (j, 0, 0)),
            pl.BlockSpec((1, B, K), lambda j: (j, 0, 0)),
        ],
        out_shape=[
            jax.ShapeDtypeStruct((B, N), jnp.float32),
            jax.ShapeDtypeStruct((NB1, B, K), jnp.float32),
            jax.ShapeDtypeStruct((NB1, B, K), jnp.int32),
        ],
    )(x, W, b2)
    cv2 = cv.transpose(1, 0, 2).reshape(B, NB1 * K)
    ci2 = ci.transpose(1, 0, 2).reshape(B, NB1 * K)
    wta = pl.pallas_call(
        _p2,
        grid=(NB2,),
        in_specs=[
            pl.BlockSpec((B, NB1 * K), lambda j: (0, 0)),
            pl.BlockSpec((B, NB1 * K), lambda j: (0, 0)),
        ],
        out_specs=pl.BlockSpec((B, BLK2), lambda j: (0, j)),
        out_shape=jax.ShapeDtypeStruct((B, N), jnp.float32),
    )(cv2, ci2)
    return (y, wta)


# speedup vs baseline: 1.9539x; 1.1528x over previous
import functools

import jax
import jax.numpy as jnp
from jax.experimental import pallas as pl
from jax.experimental.pallas import tpu as pltpu
from jax.experimental.pallas import tpu_sc as plsc

B = 128
IN = 64
N = 32768
K = 8
BLK = 2048
NB = N // BLK
NG = 16
NW = 32
RPW = B // NW
IMAX = 2**31 - 1


def _tc_pass(x_ref, w_ref, b_ref, y_ref, d_ref, g_ref, r_ref):
    j = pl.program_id(0)
    xb = x_ref[...]
    wb = w_ref[...]
    yb = jax.lax.dot_general(xb, wb, (((1,), (1,)), ((), ())),
                             preferred_element_type=jnp.float32)
    y_ref[...] = yb + b_ref[...]
    x2 = jnp.sum(xb * xb, axis=1, keepdims=True)
    w2 = jnp.sum(wb * wb, axis=1)[None, :]
    s = jnp.sqrt(jnp.maximum(x2 + w2 - 2.0 * yb, 0.0))
    d_ref[...] = s

    @pl.when(j == 0)
    def _():
        r_ref[...] = s

    @pl.when(j > 0)
    def _():
        r_ref[...] = jnp.maximum(r_ref[...], s)

    @pl.when(j == NB - 1)
    def _():
        r = r_ref[...]
        lane = jax.lax.broadcasted_iota(jnp.int32, (B, BLK), 1)
        gs = []
        for _ in range(NG):
            m = jnp.max(r, axis=1, keepdims=True)
            g = jnp.min(jnp.where(r == m, lane, IMAX), axis=1, keepdims=True)
            gs.append(g)
            r = jnp.where(lane == g, -1.0, r)
        g_ref[...] = jnp.concatenate(gs, axis=1)


_sc_mesh = plsc.VectorSubcoreMesh(core_axis_name="c", subcore_axis_name="s")


@functools.partial(
    pl.kernel,
    out_type=jax.ShapeDtypeStruct((B, N), jnp.float32),
    mesh=_sc_mesh,
    scratch_types=[
        pltpu.VMEM((N,), jnp.float32),
        pltpu.VMEM((NG,), jnp.int32),
        pltpu.VMEM((2, 128), jnp.int32),
        pltpu.VMEM((2, 128), jnp.float32),
        pltpu.SemaphoreType.DMA,
        pltpu.SemaphoreType.DMA,
    ],
    compiler_params=pltpu.CompilerParams(needs_layout_passes=False),
)
def _sc_scatter(d_hbm, g_hbm, wta_hbm, rowbuf, gbuf, idxbuf, candbuf,
                sem0, sem1):
    wid = jax.lax.axis_index("s") * 2 + jax.lax.axis_index("c")
    lane = jax.lax.iota(jnp.int32, 16)

    _dn = jax.lax.GatherDimensionNumbers(
        offset_dims=(), collapsed_slice_dims=(0,), start_index_map=(0,))

    def _shuf(v, perm):
        return jax.lax.gather(
            v, perm[:, None], _dn, slice_sizes=(1,),
            mode=jax.lax.GatherScatterMode.PROMISE_IN_BOUNDS)

    def _allmax(v):
        for sh in (8, 4, 2, 1):
            v = jnp.maximum(v, _shuf(v, (lane + sh) & 15))
        return v

    def _allmin(v):
        for sh in (8, 4, 2, 1):
            v = jnp.minimum(v, _shuf(v, (lane + sh) & 15))
        return v

    zeros = jnp.zeros((16,), jnp.float32)
    ones = jnp.full((16,), 1.0, jnp.float32)

    def _z(i, _):
        for u in range(8):
            rowbuf[pl.ds((i * 8 + u) * 16, 16)] = zeros
        return 0
    jax.lax.fori_loop(0, N // (16 * 8), _z, 0)

    def _row(t, _):
        r = wid * RPW + t
        pltpu.sync_copy(g_hbm.at[r], gbuf)
        g = gbuf[...]
        base = r * N
        for i in range(NB):
            idxbuf[i // 8, pl.ds((i % 8) * 16, 16)] = g + (i * BLK + 0) + base
        cp0 = pltpu.async_copy(d_hbm.at[idxbuf.at[0]], candbuf.at[0], sem0)
        cp1 = pltpu.async_copy(d_hbm.at[idxbuf.at[1]], candbuf.at[1], sem1)
        cp0.wait()
        cp1.wait()
        cvals = [candbuf[i // 8, pl.ds((i % 8) * 16, 16)] for i in range(NB)]
        ridx = [g + i * BLK for i in range(NB)]
        winv = None
        for k in range(K):
            bv, bi = cvals[0], ridx[0]
            for i in range(1, NB):
                better = (cvals[i] > bv) | ((cvals[i] == bv) & (ridx[i] < bi))
                bv = jnp.where(better, cvals[i], bv)
                bi = jnp.where(better, ridx[i], bi)
            m = _allmax(bv)
            win = _allmin(jnp.where(bv == m, bi, IMAX))
            winv = win if winv is None else jnp.where(lane == k, win, winv)
            for i in range(NB):
                cvals[i] = jnp.where(ridx[i] == win, -1.0, cvals[i])
        plsc.store_scatter(rowbuf, [winv], ones)
        pltpu.sync_copy(rowbuf, wta_hbm.at[r])
        plsc.store_scatter(rowbuf, [winv], zeros)
        return 0

    jax.lax.fori_loop(0, RPW, _row, 0)


def kernel(x, W, b):
    b2 = b.reshape(1, N)
    y, dist, G = pl.pallas_call(
        _tc_pass,
        grid=(NB,),
        in_specs=[
            pl.BlockSpec((B, IN), lambda j: (0, 0)),
            pl.BlockSpec((BLK, IN), lambda j: (j, 0)),
            pl.BlockSpec((1, BLK), lambda j: (0, j)),
        ],
        out_specs=[
            pl.BlockSpec((B, BLK), lambda j: (0, j)),
            pl.BlockSpec((B, BLK), lambda j: (0, j)),
            pl.BlockSpec((B, NG), lambda j: (0, 0)),
        ],
        out_shape=[
            jax.ShapeDtypeStruct((B, N), jnp.float32),
            jax.ShapeDtypeStruct((B, N), jnp.float32),
            jax.ShapeDtypeStruct((B, NG), jnp.int32),
        ],
        scratch_shapes=[pltpu.VMEM((B, BLK), jnp.float32)],
    )(x, W, b2)
    wta = _sc_scatter(dist.reshape(B * N), G)
    return (y, wta)
